# Initial kernel scaffold; baseline (speedup 1.0000x reference)
import jax
import jax.numpy as jnp
from jax.experimental import pallas as pl


def kernel(x, edge_index, W1, a_src1, a_dst1, b1, W2, a_src2, a_dst2, b2):
    raise NotImplementedError("write your pallas kernel here")


# speedup vs baseline: 21.6497x; 21.6497x over previous
import functools

import jax
import jax.numpy as jnp
from jax import lax
from jax.experimental import pallas as pl
from jax.experimental.pallas import tpu as pltpu
from jax.experimental.pallas import tpu_sc as plsc

N = 10000
E = 320000
D = 128
H1 = 8
C1 = 16
OUT = 128

LANES = 16
NC = 2
NS = 16
DH = D // NC
GPC = DH // LANES
EPT = E // NS
K = 80
NCHUNK = EPT // K
ROWS_PT = 624
TAIL = N - NS * ROWS_PT
TAIL_BASE = NS * ROWS_PT
HS_W = D + LANES

_f32 = jnp.float32


def _splat_lane(v, lane):
    idx = jnp.full((LANES, 1), lane, jnp.int32)
    dn = lax.GatherDimensionNumbers(
        offset_dims=(), collapsed_slice_dims=(0,), start_index_map=(0,))
    return lax.gather(v, idx, dn, (1,),
                      mode=lax.GatherScatterMode.PROMISE_IN_BOUNDS)


def _prep_body(nheads, x_ref, wext_ref, wadp_ref, hsrc_ref, adp_ref, m_ref):
    x = x_ref[...]
    hsrc = jnp.dot(x, wext_ref[...], preferred_element_type=_f32)
    adp = jnp.dot(x, wadp_ref[...], preferred_element_type=_f32)
    hsrc_ref[...] = hsrc
    adp_ref[...] = adp
    colh = lax.broadcasted_iota(jnp.int32, hsrc.shape, 1)
    cola = lax.broadcasted_iota(jnp.int32, adp.shape, 1)
    ninf = jnp.float32(-jnp.inf)
    asmax = jnp.max(jnp.where((colh >= D) & (colh < D + nheads), hsrc, ninf))
    admax = jnp.max(jnp.where(cola < nheads, adp, ninf))
    m = jnp.maximum(asmax + admax, 0.0)
    m_ref[...] = jnp.full((1, D), m, _f32)


def _prep_call(nheads, x, wext, wadp):
    return pl.pallas_call(
        functools.partial(_prep_body, nheads),
        out_shape=(
            jax.ShapeDtypeStruct((N, HS_W), _f32),
            jax.ShapeDtypeStruct((N, LANES), _f32),
            jax.ShapeDtypeStruct((1, D), _f32),
        ),
    )(x, wext, wadp)


def _mid_body(acc_ref, den_ref, b1_ref, w2ext_ref, w2adp_ref,
              hsrc_ref, adp_ref, m_ref):
    num = jnp.concatenate([acc_ref[0], acc_ref[1]], axis=1)
    den = den_ref[0]
    srow = lax.broadcasted_iota(jnp.int32, (LANES, D), 0)
    scol = lax.broadcasted_iota(jnp.int32, (LANES, D), 1)
    sel = (srow == scol // C1).astype(_f32)
    dexp = jnp.dot(den, sel, preferred_element_type=_f32)
    x2 = num / (dexp + 1e-16) + b1_ref[...]
    x2 = jnp.where(x2 > 0, x2, jnp.exp(x2) - 1.0)
    hsrc = jnp.dot(x2, w2ext_ref[...], preferred_element_type=_f32)
    adp = jnp.dot(x2, w2adp_ref[...], preferred_element_type=_f32)
    hsrc_ref[...] = hsrc
    adp_ref[...] = adp
    colh = lax.broadcasted_iota(jnp.int32, hsrc.shape, 1)
    cola = lax.broadcasted_iota(jnp.int32, adp.shape, 1)
    ninf = jnp.float32(-jnp.inf)
    asmax = jnp.max(jnp.where(colh == D, hsrc, ninf))
    admax = jnp.max(jnp.where(cola == 0, adp, ninf))
    m = jnp.maximum(asmax + admax, 0.0)
    m_ref[...] = jnp.full((1, D), m, _f32)


def _mid_call(acc, den, b1r, w2ext, w2adp):
    return pl.pallas_call(
        _mid_body,
        out_shape=(
            jax.ShapeDtypeStruct((N, HS_W), _f32),
            jax.ShapeDtypeStruct((N, LANES), _f32),
            jax.ShapeDtypeStruct((1, D), _f32),
        ),
    )(acc, den, b1r, w2ext, w2adp)


def _final_body(acc_ref, den_ref, b2_ref, out_ref):
    num = jnp.concatenate([acc_ref[0], acc_ref[1]], axis=1)
    den = den_ref[0]
    srow = lax.broadcasted_iota(jnp.int32, (LANES, D), 0)
    sel = (srow == 0).astype(_f32)
    dexp = jnp.dot(den, sel, preferred_element_type=_f32)
    out_ref[...] = num / (dexp + 1e-16) + b2_ref[...]


def _final_call(acc, den, b2r):
    return pl.pallas_call(
        _final_body,
        out_shape=jax.ShapeDtypeStruct((N, OUT), _f32),
    )(acc, den, b2r)


def _edge_body(nheads, ei_ref, hsrc_ref, adp_ref, m_ref,
               acc_out, den_out,
               srcall, dstall, hrows, adrows, wvals, wrows, mvec,
               accs, dens, sem_a, sem_b):
    cid = lax.axis_index("c")
    sid = lax.axis_index("s")
    rbase = sid * ROWS_PT
    cbase = cid * GPC

    def _zrow(i, _):
        for g in range(GPC):
            wrows[i, pl.ds(g * LANES, LANES)] = jnp.zeros((LANES,), _f32)
        wvals[i, :] = jnp.zeros((LANES,), _f32)
        return 0
    lax.fori_loop(0, K, _zrow, 0)
    for r in range(ROWS_PT // K):
        pltpu.sync_copy(wrows, accs.at[pl.ds(rbase + r * K, K)])
        pltpu.sync_copy(wvals, dens.at[pl.ds(rbase + r * K, K)])
    rleft = ROWS_PT - (ROWS_PT // K) * K
    if rleft:
        off = rbase + (ROWS_PT // K) * K
        pltpu.sync_copy(wrows.at[pl.ds(0, rleft)], accs.at[pl.ds(off, rleft)])
        pltpu.sync_copy(wvals.at[pl.ds(0, rleft)], dens.at[pl.ds(off, rleft)])

    @pl.when(sid == 0)
    def _zero_tail():
        pltpu.sync_copy(wrows.at[pl.ds(0, TAIL)],
                        accs.at[pl.ds(TAIL_BASE, TAIL)])
        pltpu.sync_copy(wvals.at[pl.ds(0, TAIL)],
                        dens.at[pl.ds(TAIL_BASE, TAIL)])
    plsc.subcore_barrier()

    pltpu.sync_copy(ei_ref.at[0, sid], srcall)
    pltpu.sync_copy(ei_ref.at[1, sid], dstall)
    pltpu.sync_copy(m_ref.at[0, pl.ds(0, LANES)], mvec)

    def _chunk(j, _):
        cp_h = pltpu.async_copy(hsrc_ref.at[srcall.at[j]], hrows, sem_a)
        cp_a = pltpu.async_copy(adp_ref.at[dstall.at[j]], adrows, sem_b)
        cp_h.wait()
        cp_a.wait()
        mv = mvec[...]

        def _edge(k, _):
            e16 = hrows[k, pl.ds(D, LANES)] + adrows[k, :]
            e16 = jnp.where(e16 > 0, e16, 0.2 * e16)
            w16 = jnp.exp(e16 - mv)
            wvals[k, :] = w16
            if nheads == 1:
                ws = _splat_lane(w16, 0)
                for g in range(GPC):
                    col = cbase * LANES + g * LANES
                    wrows[k, pl.ds(g * LANES, LANES)] = (
                        hrows[k, pl.ds(col, LANES)] * ws)
            else:
                for g in range(GPC):
                    col = cbase * LANES + g * LANES
                    ws = _splat_lane(w16, cbase + g)
                    wrows[k, pl.ds(g * LANES, LANES)] = (
                        hrows[k, pl.ds(col, LANES)] * ws)
            return 0
        lax.fori_loop(0, K, _edge, 0)

        pltpu.sync_copy(wrows, accs.at[dstall.at[j]], add=True)
        pltpu.sync_copy(wvals, dens.at[dstall.at[j]], add=True)
        return 0
    lax.fori_loop(0, NCHUNK, _chunk, 0)

    plsc.subcore_barrier()
    pltpu.sync_copy(accs.at[pl.ds(rbase, ROWS_PT)],
                    acc_out.at[cid, pl.ds(rbase, ROWS_PT)])
    pltpu.sync_copy(dens.at[pl.ds(rbase, ROWS_PT)],
                    den_out.at[cid, pl.ds(rbase, ROWS_PT)])

    @pl.when(sid == 0)
    def _pub_tail():
        pltpu.sync_copy(accs.at[pl.ds(TAIL_BASE, TAIL)],
                        acc_out.at[cid, pl.ds(TAIL_BASE, TAIL)])
        pltpu.sync_copy(dens.at[pl.ds(TAIL_BASE, TAIL)],
                        den_out.at[cid, pl.ds(TAIL_BASE, TAIL)])


def _edge_call(nheads, ei4, hsrc, adp, m):
    mesh = plsc.VectorSubcoreMesh(
        core_axis_name="c", subcore_axis_name="s",
        num_cores=NC, num_subcores=NS)
    kern = pl.kernel(
        functools.partial(_edge_body, nheads),
        out_type=(
            jax.ShapeDtypeStruct((NC, N, DH), _f32),
            jax.ShapeDtypeStruct((NC, N, LANES), _f32),
        ),
        mesh=mesh,
        compiler_params=pltpu.CompilerParams(use_tc_tiling_on_sc=False),
        scratch_types=(
            pltpu.VMEM((NCHUNK, K), jnp.int32),
            pltpu.VMEM((NCHUNK, K), jnp.int32),
            pltpu.VMEM((K, HS_W), _f32),
            pltpu.VMEM((K, LANES), _f32),
            pltpu.VMEM((K, LANES), _f32),
            pltpu.VMEM((K, DH), _f32),
            pltpu.VMEM((LANES,), _f32),
            pltpu.VMEM_SHARED((N, DH), _f32),
            pltpu.VMEM_SHARED((N, LANES), _f32),
            pltpu.SemaphoreType.DMA,
            pltpu.SemaphoreType.DMA,
        ),
    )
    return kern(ei4, hsrc, adp, m)


def kernel(x, edge_index, W1, a_src1, a_dst1, b1, W2, a_src2, a_dst2, b2):
    ar = jnp.arange(D)
    A1s = jnp.zeros((D, H1), _f32).at[ar, ar // C1].set(a_src1.reshape(-1))
    A1d = jnp.zeros((D, H1), _f32).at[ar, ar // C1].set(a_dst1.reshape(-1))
    zpad = jnp.zeros((D, LANES - H1), _f32)
    w1ext = jnp.concatenate([W1, W1 @ A1s, zpad], axis=1)
    w1adp = jnp.concatenate([W1 @ A1d, zpad], axis=1)
    zpad2 = jnp.zeros((D, LANES - 1), _f32)
    w2ext = jnp.concatenate([W2, W2 @ a_src2.T, zpad2], axis=1)
    w2adp = jnp.concatenate([W2 @ a_dst2.T, zpad2], axis=1)
    ei4 = edge_index.reshape(2, NS, NCHUNK, K)
    b1r = b1.reshape(1, D)
    b2r = b2.reshape(1, OUT)

    hsrc1, adp1, m1 = _prep_call(H1, x, w1ext, w1adp)
    acc1, den1 = _edge_call(H1, ei4, hsrc1, adp1, m1)
    hsrc2, adp2, m2 = _mid_call(acc1, den1, b1r, w2ext, w2adp)
    acc2, den2 = _edge_call(1, ei4, hsrc2, adp2, m2)
    return _final_call(acc2, den2, b2r)


# speedup vs baseline: 29.4655x; 1.3610x over previous
import functools

import jax
import jax.numpy as jnp
from jax import lax
from jax.experimental import pallas as pl
from jax.experimental.pallas import tpu as pltpu
from jax.experimental.pallas import tpu_sc as plsc

N = 10000
E = 320000
D = 128
H1 = 8
C1 = 16
OUT = 128

LANES = 16
NC = 2
NS = 16
DH = D // NC
GPC = DH // LANES
EPT = E // NS
K = 80
NCHUNK = EPT // K
ROWS_PT = 624
TAIL = N - NS * ROWS_PT
TAIL_BASE = NS * ROWS_PT
HS_W = D + LANES

_f32 = jnp.float32


def _splat_lane(v, lane):
    idx = jnp.full((LANES, 1), lane, jnp.int32)
    dn = lax.GatherDimensionNumbers(
        offset_dims=(), collapsed_slice_dims=(0,), start_index_map=(0,))
    return lax.gather(v, idx, dn, (1,),
                      mode=lax.GatherScatterMode.PROMISE_IN_BOUNDS)


def _prep_body(nheads, x_ref, wext_ref, wadp_ref, hsrc_ref, adp_ref, m_ref):
    x = x_ref[...]
    hsrc = jnp.dot(x, wext_ref[...], preferred_element_type=_f32)
    adp = jnp.dot(x, wadp_ref[...], preferred_element_type=_f32)
    hsrc_ref[...] = hsrc
    adp_ref[...] = adp
    colh = lax.broadcasted_iota(jnp.int32, hsrc.shape, 1)
    cola = lax.broadcasted_iota(jnp.int32, adp.shape, 1)
    ninf = jnp.float32(-jnp.inf)
    asmax = jnp.max(jnp.where((colh >= D) & (colh < D + nheads), hsrc, ninf))
    admax = jnp.max(jnp.where(cola < nheads, adp, ninf))
    m = jnp.maximum(asmax + admax, 0.0)
    m_ref[...] = jnp.full((1, D), m, _f32)


def _prep_call(nheads, x, wext, wadp):
    return pl.pallas_call(
        functools.partial(_prep_body, nheads),
        out_shape=(
            jax.ShapeDtypeStruct((N, HS_W), _f32),
            jax.ShapeDtypeStruct((N, LANES), _f32),
            jax.ShapeDtypeStruct((1, D), _f32),
        ),
    )(x, wext, wadp)


def _mid_body(acc_ref, den_ref, b1_ref, w2ext_ref, w2adp_ref,
              hsrc_ref, adp_ref, m_ref):
    num = jnp.concatenate([acc_ref[0], acc_ref[1]], axis=1)
    den = den_ref[0]
    srow = lax.broadcasted_iota(jnp.int32, (LANES, D), 0)
    scol = lax.broadcasted_iota(jnp.int32, (LANES, D), 1)
    sel = (srow == scol // C1).astype(_f32)
    dexp = jnp.dot(den, sel, preferred_element_type=_f32)
    x2 = num / (dexp + 1e-16) + b1_ref[...]
    x2 = jnp.where(x2 > 0, x2, jnp.exp(x2) - 1.0)
    hsrc = jnp.dot(x2, w2ext_ref[...], preferred_element_type=_f32)
    adp = jnp.dot(x2, w2adp_ref[...], preferred_element_type=_f32)
    hsrc_ref[...] = hsrc
    adp_ref[...] = adp
    colh = lax.broadcasted_iota(jnp.int32, hsrc.shape, 1)
    cola = lax.broadcasted_iota(jnp.int32, adp.shape, 1)
    ninf = jnp.float32(-jnp.inf)
    asmax = jnp.max(jnp.where(colh == D, hsrc, ninf))
    admax = jnp.max(jnp.where(cola == 0, adp, ninf))
    m = jnp.maximum(asmax + admax, 0.0)
    m_ref[...] = jnp.full((1, D), m, _f32)


def _mid_call(acc, den, b1r, w2ext, w2adp):
    return pl.pallas_call(
        _mid_body,
        out_shape=(
            jax.ShapeDtypeStruct((N, HS_W), _f32),
            jax.ShapeDtypeStruct((N, LANES), _f32),
            jax.ShapeDtypeStruct((1, D), _f32),
        ),
    )(acc, den, b1r, w2ext, w2adp)


def _final_body(acc_ref, den_ref, b2_ref, out_ref):
    num = jnp.concatenate([acc_ref[0], acc_ref[1]], axis=1)
    den = den_ref[0]
    srow = lax.broadcasted_iota(jnp.int32, (LANES, D), 0)
    sel = (srow == 0).astype(_f32)
    dexp = jnp.dot(den, sel, preferred_element_type=_f32)
    out_ref[...] = num / (dexp + 1e-16) + b2_ref[...]


def _final_call(acc, den, b2r):
    return pl.pallas_call(
        _final_body,
        out_shape=jax.ShapeDtypeStruct((N, OUT), _f32),
    )(acc, den, b2r)


def _edge_body(nheads, ei_ref, hsrc_ref, adp_ref, m_ref,
               acc_out, den_out,
               srcall, dstall, hrows0, hrows1, adrows0, adrows1,
               wvals, wrows, mvec,
               accs, dens, sem_h0, sem_h1, sem_a0, sem_a1):
    hrows_b = (hrows0, hrows1)
    adrows_b = (adrows0, adrows1)
    sem_h = (sem_h0, sem_h1)
    sem_a = (sem_a0, sem_a1)
    cid = lax.axis_index("c")
    sid = lax.axis_index("s")
    rbase = sid * ROWS_PT
    cbase = cid * GPC

    def _zrow(i, _):
        for g in range(GPC):
            wrows[i, pl.ds(g * LANES, LANES)] = jnp.zeros((LANES,), _f32)
        wvals[i, :] = jnp.zeros((LANES,), _f32)
        return 0
    lax.fori_loop(0, K, _zrow, 0)
    for r in range(ROWS_PT // K):
        pltpu.sync_copy(wrows, accs.at[pl.ds(rbase + r * K, K)])
        pltpu.sync_copy(wvals, dens.at[pl.ds(rbase + r * K, K)])
    rleft = ROWS_PT - (ROWS_PT // K) * K
    if rleft:
        off = rbase + (ROWS_PT // K) * K
        pltpu.sync_copy(wrows.at[pl.ds(0, rleft)], accs.at[pl.ds(off, rleft)])
        pltpu.sync_copy(wvals.at[pl.ds(0, rleft)], dens.at[pl.ds(off, rleft)])

    @pl.when(sid == 0)
    def _zero_tail():
        pltpu.sync_copy(wrows.at[pl.ds(0, TAIL)],
                        accs.at[pl.ds(TAIL_BASE, TAIL)])
        pltpu.sync_copy(wvals.at[pl.ds(0, TAIL)],
                        dens.at[pl.ds(TAIL_BASE, TAIL)])
    plsc.subcore_barrier()

    pltpu.sync_copy(ei_ref.at[0, sid], srcall)
    pltpu.sync_copy(ei_ref.at[1, sid], dstall)
    pltpu.sync_copy(m_ref.at[0, pl.ds(0, LANES)], mvec)

    mv0 = mvec[...]

    for b in range(2):
        pltpu.async_copy(hsrc_ref.at[srcall.at[b]], hrows_b[b], sem_h[b])
        pltpu.async_copy(adp_ref.at[dstall.at[b]], adrows_b[b], sem_a[b])

    def _super(jj, _):
        for b in range(2):
            j = jj * 2 + b
            hrows = hrows_b[b]
            adrows = adrows_b[b]
            pltpu.make_async_copy(hsrc_ref.at[srcall.at[j]],
                                  hrows, sem_h[b]).wait()
            pltpu.make_async_copy(adp_ref.at[dstall.at[j]],
                                  adrows, sem_a[b]).wait()

            def _edge(k, _):
                e16 = hrows[k, pl.ds(D, LANES)] + adrows[k, :]
                e16 = jnp.where(e16 > 0, e16, 0.2 * e16)
                w16 = jnp.exp(e16 - mv0)
                wvals[k, :] = w16
                if nheads == 1:
                    ws = _splat_lane(w16, 0)
                    for g in range(GPC):
                        col = cbase * LANES + g * LANES
                        wrows[k, pl.ds(g * LANES, LANES)] = (
                            hrows[k, pl.ds(col, LANES)] * ws)
                else:
                    for g in range(GPC):
                        col = cbase * LANES + g * LANES
                        ws = _splat_lane(w16, cbase + g)
                        wrows[k, pl.ds(g * LANES, LANES)] = (
                            hrows[k, pl.ds(col, LANES)] * ws)
                return 0
            lax.fori_loop(0, K, _edge, 0, unroll=4)

            jn = j + 2

            @pl.when(jn < NCHUNK)
            def _next():
                pltpu.async_copy(hsrc_ref.at[srcall.at[jn]],
                                 hrows, sem_h[b])
                pltpu.async_copy(adp_ref.at[dstall.at[jn]],
                                 adrows, sem_a[b])

            pltpu.sync_copy(wrows, accs.at[dstall.at[j]], add=True)
            pltpu.sync_copy(wvals, dens.at[dstall.at[j]], add=True)
        return 0
    lax.fori_loop(0, NCHUNK // 2, _super, 0)

    plsc.subcore_barrier()
    pltpu.sync_copy(accs.at[pl.ds(rbase, ROWS_PT)],
                    acc_out.at[cid, pl.ds(rbase, ROWS_PT)])
    pltpu.sync_copy(dens.at[pl.ds(rbase, ROWS_PT)],
                    den_out.at[cid, pl.ds(rbase, ROWS_PT)])

    @pl.when(sid == 0)
    def _pub_tail():
        pltpu.sync_copy(accs.at[pl.ds(TAIL_BASE, TAIL)],
                        acc_out.at[cid, pl.ds(TAIL_BASE, TAIL)])
        pltpu.sync_copy(dens.at[pl.ds(TAIL_BASE, TAIL)],
                        den_out.at[cid, pl.ds(TAIL_BASE, TAIL)])


def _edge_call(nheads, ei4, hsrc, adp, m):
    mesh = plsc.VectorSubcoreMesh(
        core_axis_name="c", subcore_axis_name="s",
        num_cores=NC, num_subcores=NS)
    kern = pl.kernel(
        functools.partial(_edge_body, nheads),
        out_type=(
            jax.ShapeDtypeStruct((NC, N, DH), _f32),
            jax.ShapeDtypeStruct((NC, N, LANES), _f32),
        ),
        mesh=mesh,
        compiler_params=pltpu.CompilerParams(use_tc_tiling_on_sc=False),
        scratch_types=(
            pltpu.VMEM((NCHUNK, K), jnp.int32),
            pltpu.VMEM((NCHUNK, K), jnp.int32),
            pltpu.VMEM((K, HS_W), _f32),
            pltpu.VMEM((K, HS_W), _f32),
            pltpu.VMEM((K, LANES), _f32),
            pltpu.VMEM((K, LANES), _f32),
            pltpu.VMEM((K, LANES), _f32),
            pltpu.VMEM((K, DH), _f32),
            pltpu.VMEM((LANES,), _f32),
            pltpu.VMEM_SHARED((N, DH), _f32),
            pltpu.VMEM_SHARED((N, LANES), _f32),
            pltpu.SemaphoreType.DMA,
            pltpu.SemaphoreType.DMA,
            pltpu.SemaphoreType.DMA,
            pltpu.SemaphoreType.DMA,
        ),
    )
    return kern(ei4, hsrc, adp, m)


def kernel(x, edge_index, W1, a_src1, a_dst1, b1, W2, a_src2, a_dst2, b2):
    ar = jnp.arange(D)
    A1s = jnp.zeros((D, H1), _f32).at[ar, ar // C1].set(a_src1.reshape(-1))
    A1d = jnp.zeros((D, H1), _f32).at[ar, ar // C1].set(a_dst1.reshape(-1))
    zpad = jnp.zeros((D, LANES - H1), _f32)
    w1ext = jnp.concatenate([W1, W1 @ A1s, zpad], axis=1)
    w1adp = jnp.concatenate([W1 @ A1d, zpad], axis=1)
    zpad2 = jnp.zeros((D, LANES - 1), _f32)
    w2ext = jnp.concatenate([W2, W2 @ a_src2.T, zpad2], axis=1)
    w2adp = jnp.concatenate([W2 @ a_dst2.T, zpad2], axis=1)
    ei4 = edge_index.reshape(2, NS, NCHUNK, K)
    b1r = b1.reshape(1, D)
    b2r = b2.reshape(1, OUT)

    hsrc1, adp1, m1 = _prep_call(H1, x, w1ext, w1adp)
    acc1, den1 = _edge_call(H1, ei4, hsrc1, adp1, m1)
    hsrc2, adp2, m2 = _mid_call(acc1, den1, b1r, w2ext, w2adp)
    acc2, den2 = _edge_call(1, ei4, hsrc2, adp2, m2)
    return _final_call(acc2, den2, b2r)


# speedup vs baseline: 33.9394x; 1.1518x over previous
import functools

import jax
import jax.numpy as jnp
from jax import lax
from jax.experimental import pallas as pl
from jax.experimental.pallas import tpu as pltpu
from jax.experimental.pallas import tpu_sc as plsc

N = 10000
E = 320000
D = 128
H1 = 8
C1 = 16
OUT = 128

LANES = 16
NC = 2
NS = 16
DH = D // NC
GPC = DH // LANES
EPT = E // NS
K = 80
NCHUNK = EPT // K
ROWS_PT = 624
TAIL = N - NS * ROWS_PT
TAIL_BASE = NS * ROWS_PT
HS_W = D + LANES

_f32 = jnp.float32


def _splat_lane(v, lane):
    idx = jnp.full((LANES, 1), lane, jnp.int32)
    dn = lax.GatherDimensionNumbers(
        offset_dims=(), collapsed_slice_dims=(0,), start_index_map=(0,))
    return lax.gather(v, idx, dn, (1,),
                      mode=lax.GatherScatterMode.PROMISE_IN_BOUNDS)


def _prep_body(nheads, x_ref, wext_ref, wadp_ref, hsrc_ref, adp_ref, m_ref):
    x = x_ref[...]
    hsrc = jnp.dot(x, wext_ref[...], preferred_element_type=_f32)
    adp = jnp.dot(x, wadp_ref[...], preferred_element_type=_f32)
    hsrc_ref[...] = hsrc
    adp_ref[...] = adp
    colh = lax.broadcasted_iota(jnp.int32, hsrc.shape, 1)
    cola = lax.broadcasted_iota(jnp.int32, adp.shape, 1)
    ninf = jnp.float32(-jnp.inf)
    asmax = jnp.max(jnp.where((colh >= D) & (colh < D + nheads), hsrc, ninf))
    admax = jnp.max(jnp.where(cola < nheads, adp, ninf))
    m = jnp.maximum(asmax + admax, 0.0)
    m_ref[...] = jnp.full((1, D), m, _f32)


def _prep_call(nheads, x, wext, wadp):
    return pl.pallas_call(
        functools.partial(_prep_body, nheads),
        out_shape=(
            jax.ShapeDtypeStruct((N, HS_W), _f32),
            jax.ShapeDtypeStruct((N, LANES), _f32),
            jax.ShapeDtypeStruct((1, D), _f32),
        ),
    )(x, wext, wadp)


def _mid_body(acc_ref, den_ref, b1_ref, w2ext_ref, w2adp_ref,
              hsrc_ref, adp_ref, m_ref):
    num = jnp.concatenate([acc_ref[0], acc_ref[1]], axis=1)
    den = den_ref[0]
    srow = lax.broadcasted_iota(jnp.int32, (LANES, D), 0)
    scol = lax.broadcasted_iota(jnp.int32, (LANES, D), 1)
    sel = (srow == scol // C1).astype(_f32)
    dexp = jnp.dot(den, sel, preferred_element_type=_f32)
    x2 = num / (dexp + 1e-16) + b1_ref[...]
    x2 = jnp.where(x2 > 0, x2, jnp.exp(x2) - 1.0)
    hsrc = jnp.dot(x2, w2ext_ref[...], preferred_element_type=_f32)
    adp = jnp.dot(x2, w2adp_ref[...], preferred_element_type=_f32)
    hsrc_ref[...] = hsrc
    adp_ref[...] = adp
    colh = lax.broadcasted_iota(jnp.int32, hsrc.shape, 1)
    cola = lax.broadcasted_iota(jnp.int32, adp.shape, 1)
    ninf = jnp.float32(-jnp.inf)
    asmax = jnp.max(jnp.where(colh == D, hsrc, ninf))
    admax = jnp.max(jnp.where(cola == 0, adp, ninf))
    m = jnp.maximum(asmax + admax, 0.0)
    m_ref[...] = jnp.full((1, D), m, _f32)


def _mid_call(acc, den, b1r, w2ext, w2adp):
    return pl.pallas_call(
        _mid_body,
        out_shape=(
            jax.ShapeDtypeStruct((N, HS_W), _f32),
            jax.ShapeDtypeStruct((N, LANES), _f32),
            jax.ShapeDtypeStruct((1, D), _f32),
        ),
    )(acc, den, b1r, w2ext, w2adp)


def _final_body(acc_ref, den_ref, b2_ref, out_ref):
    num = jnp.concatenate([acc_ref[0], acc_ref[1]], axis=1)
    den = den_ref[0]
    srow = lax.broadcasted_iota(jnp.int32, (LANES, D), 0)
    sel = (srow == 0).astype(_f32)
    dexp = jnp.dot(den, sel, preferred_element_type=_f32)
    out_ref[...] = num / (dexp + 1e-16) + b2_ref[...]


def _final_call(acc, den, b2r):
    return pl.pallas_call(
        _final_body,
        out_shape=jax.ShapeDtypeStruct((N, OUT), _f32),
    )(acc, den, b2r)


def _edge_body(nheads, ei_ref, hsrc_ref, adp_ref, m_ref,
               acc_out, den_out,
               srcall, dstall, hrows0, hrows1, adrows0, adrows1,
               wvals0, wvals1, wrows0, wrows1, mvec,
               accs, dens, sem_h0, sem_h1, sem_a0, sem_a1, sem_s0, sem_s1):
    hrows_b = (hrows0, hrows1)
    adrows_b = (adrows0, adrows1)
    wvals_b = (wvals0, wvals1)
    wrows_b = (wrows0, wrows1)
    sem_h = (sem_h0, sem_h1)
    sem_a = (sem_a0, sem_a1)
    sem_s = (sem_s0, sem_s1)
    wrows = wrows0
    wvals = wvals0
    cid = lax.axis_index("c")
    sid = lax.axis_index("s")
    rbase = sid * ROWS_PT
    cbase = cid * GPC

    def _zrow(i, _):
        for g in range(GPC):
            wrows[i, pl.ds(g * LANES, LANES)] = jnp.zeros((LANES,), _f32)
        wvals[i, :] = jnp.zeros((LANES,), _f32)
        return 0
    lax.fori_loop(0, K, _zrow, 0)
    for r in range(ROWS_PT // K):
        pltpu.sync_copy(wrows, accs.at[pl.ds(rbase + r * K, K)])
        pltpu.sync_copy(wvals, dens.at[pl.ds(rbase + r * K, K)])
    rleft = ROWS_PT - (ROWS_PT // K) * K
    if rleft:
        off = rbase + (ROWS_PT // K) * K
        pltpu.sync_copy(wrows.at[pl.ds(0, rleft)], accs.at[pl.ds(off, rleft)])
        pltpu.sync_copy(wvals.at[pl.ds(0, rleft)], dens.at[pl.ds(off, rleft)])

    @pl.when(sid == 0)
    def _zero_tail():
        pltpu.sync_copy(wrows.at[pl.ds(0, TAIL)],
                        accs.at[pl.ds(TAIL_BASE, TAIL)])
        pltpu.sync_copy(wvals.at[pl.ds(0, TAIL)],
                        dens.at[pl.ds(TAIL_BASE, TAIL)])
    plsc.subcore_barrier()

    pltpu.sync_copy(ei_ref.at[0, sid], srcall)
    pltpu.sync_copy(ei_ref.at[1, sid], dstall)
    pltpu.sync_copy(m_ref.at[0, pl.ds(0, LANES)], mvec)

    mv0 = mvec[...]

    for b in range(2):
        pltpu.async_copy(hsrc_ref.at[srcall.at[b]], hrows_b[b], sem_h[b])
        pltpu.async_copy(adp_ref.at[dstall.at[b]], adrows_b[b], sem_a[b])

    def _super(jj, _):
        for b in range(2):
            j = jj * 2 + b
            hrows = hrows_b[b]
            adrows = adrows_b[b]
            wrows = wrows_b[b]
            wvals = wvals_b[b]
            pltpu.make_async_copy(hsrc_ref.at[srcall.at[j]],
                                  hrows, sem_h[b]).wait()
            pltpu.make_async_copy(adp_ref.at[dstall.at[j]],
                                  adrows, sem_a[b]).wait()

            @pl.when(jj > 0)
            def _drain():
                pltpu.make_async_copy(wrows, accs.at[dstall.at[j]],
                                      sem_s[b]).wait()
                pltpu.make_async_copy(wvals, dens.at[dstall.at[j]],
                                      sem_s[b]).wait()

            def _edge(k, _):
                e16 = hrows[k, pl.ds(D, LANES)] + adrows[k, :]
                e16 = jnp.where(e16 > 0, e16, 0.2 * e16)
                w16 = jnp.exp(e16 - mv0)
                wvals[k, :] = w16
                if nheads == 1:
                    ws = _splat_lane(w16, 0)
                    for g in range(GPC):
                        col = cbase * LANES + g * LANES
                        wrows[k, pl.ds(g * LANES, LANES)] = (
                            hrows[k, pl.ds(col, LANES)] * ws)
                else:
                    for g in range(GPC):
                        col = cbase * LANES + g * LANES
                        ws = _splat_lane(w16, cbase + g)
                        wrows[k, pl.ds(g * LANES, LANES)] = (
                            hrows[k, pl.ds(col, LANES)] * ws)
                return 0
            lax.fori_loop(0, K, _edge, 0, unroll=8)

            jn = j + 2

            @pl.when(jn < NCHUNK)
            def _next():
                pltpu.async_copy(hsrc_ref.at[srcall.at[jn]],
                                 hrows, sem_h[b])
                pltpu.async_copy(adp_ref.at[dstall.at[jn]],
                                 adrows, sem_a[b])

            pltpu.async_copy(wrows, accs.at[dstall.at[j]], sem_s[b],
                             add=True)
            pltpu.async_copy(wvals, dens.at[dstall.at[j]], sem_s[b],
                             add=True)
        return 0
    lax.fori_loop(0, NCHUNK // 2, _super, 0)

    for b in range(2):
        pltpu.make_async_copy(wrows_b[b], accs.at[dstall.at[b]],
                              sem_s[b]).wait()
        pltpu.make_async_copy(wvals_b[b], dens.at[dstall.at[b]],
                              sem_s[b]).wait()

    plsc.subcore_barrier()
    pltpu.sync_copy(accs.at[pl.ds(rbase, ROWS_PT)],
                    acc_out.at[cid, pl.ds(rbase, ROWS_PT)])
    pltpu.sync_copy(dens.at[pl.ds(rbase, ROWS_PT)],
                    den_out.at[cid, pl.ds(rbase, ROWS_PT)])

    @pl.when(sid == 0)
    def _pub_tail():
        pltpu.sync_copy(accs.at[pl.ds(TAIL_BASE, TAIL)],
                        acc_out.at[cid, pl.ds(TAIL_BASE, TAIL)])
        pltpu.sync_copy(dens.at[pl.ds(TAIL_BASE, TAIL)],
                        den_out.at[cid, pl.ds(TAIL_BASE, TAIL)])


def _edge_call(nheads, ei4, hsrc, adp, m):
    mesh = plsc.VectorSubcoreMesh(
        core_axis_name="c", subcore_axis_name="s",
        num_cores=NC, num_subcores=NS)
    kern = pl.kernel(
        functools.partial(_edge_body, nheads),
        out_type=(
            jax.ShapeDtypeStruct((NC, N, DH), _f32),
            jax.ShapeDtypeStruct((NC, N, LANES), _f32),
        ),
        mesh=mesh,
        compiler_params=pltpu.CompilerParams(use_tc_tiling_on_sc=False),
        scratch_types=(
            pltpu.VMEM((NCHUNK, K), jnp.int32),
            pltpu.VMEM((NCHUNK, K), jnp.int32),
            pltpu.VMEM((K, HS_W), _f32),
            pltpu.VMEM((K, HS_W), _f32),
            pltpu.VMEM((K, LANES), _f32),
            pltpu.VMEM((K, LANES), _f32),
            pltpu.VMEM((K, LANES), _f32),
            pltpu.VMEM((K, LANES), _f32),
            pltpu.VMEM((K, DH), _f32),
            pltpu.VMEM((K, DH), _f32),
            pltpu.VMEM((LANES,), _f32),
            pltpu.VMEM_SHARED((N, DH), _f32),
            pltpu.VMEM_SHARED((N, LANES), _f32),
            pltpu.SemaphoreType.DMA,
            pltpu.SemaphoreType.DMA,
            pltpu.SemaphoreType.DMA,
            pltpu.SemaphoreType.DMA,
            pltpu.SemaphoreType.DMA,
            pltpu.SemaphoreType.DMA,
        ),
    )
    return kern(ei4, hsrc, adp, m)


def kernel(x, edge_index, W1, a_src1, a_dst1, b1, W2, a_src2, a_dst2, b2):
    ar = jnp.arange(D)
    A1s = jnp.zeros((D, H1), _f32).at[ar, ar // C1].set(a_src1.reshape(-1))
    A1d = jnp.zeros((D, H1), _f32).at[ar, ar // C1].set(a_dst1.reshape(-1))
    zpad = jnp.zeros((D, LANES - H1), _f32)
    w1ext = jnp.concatenate([W1, W1 @ A1s, zpad], axis=1)
    w1adp = jnp.concatenate([W1 @ A1d, zpad], axis=1)
    zpad2 = jnp.zeros((D, LANES - 1), _f32)
    w2ext = jnp.concatenate([W2, W2 @ a_src2.T, zpad2], axis=1)
    w2adp = jnp.concatenate([W2 @ a_dst2.T, zpad2], axis=1)
    ei4 = edge_index.reshape(2, NS, NCHUNK, K)
    b1r = b1.reshape(1, D)
    b2r = b2.reshape(1, OUT)

    hsrc1, adp1, m1 = _prep_call(H1, x, w1ext, w1adp)
    acc1, den1 = _edge_call(H1, ei4, hsrc1, adp1, m1)
    hsrc2, adp2, m2 = _mid_call(acc1, den1, b1r, w2ext, w2adp)
    acc2, den2 = _edge_call(1, ei4, hsrc2, adp2, m2)
    return _final_call(acc2, den2, b2r)
